# baseline (speedup 1.0000x reference)
import functools

import jax
import jax.numpy as jnp
from jax import lax
from jax.experimental import pallas as pl
from jax.experimental.pallas import tpu as pltpu
from jax.experimental.pallas import tpu_sc as plsc

VOCAB = 100000
EMB = 64
B = 1024
MAX_NORM = 1.0


NVF = 2048


def _fmt_body(t_ref, out_ref):
    t3 = t_ref[...].reshape(NVF // 2, 2, EMB)
    out_ref[...] = jnp.concatenate([t3[:, 0, :], t3[:, 1, :]], axis=1)


def _make_fmt():
    grid = (pl.cdiv(VOCAB, NVF),)
    return pl.pallas_call(
        _fmt_body,
        grid=grid,
        in_specs=[pl.BlockSpec((NVF, EMB), lambda j: (j, 0))],
        out_specs=pl.BlockSpec((NVF // 2, 2 * EMB), lambda j: (j, 0)),
        out_shape=jax.ShapeDtypeStruct((VOCAB // 2, 2 * EMB), jnp.float32),
        compiler_params=pltpu.CompilerParams(
            dimension_semantics=("arbitrary",),
        ),
    )


_fmt = _make_fmt()


def _make_sc_gather():
    info = plsc.get_sparse_core_info()
    nc, ns = info.num_cores, info.num_subcores
    nw = nc * ns
    b_per_w = B // nw
    mesh = plsc.VectorSubcoreMesh(core_axis_name="c", subcore_axis_name="s")

    @functools.partial(
        pl.kernel,
        mesh=mesh,
        out_type=jax.ShapeDtypeStruct((B, 2 * EMB), jnp.float32),
        scratch_types=[
            pltpu.VMEM((b_per_w,), jnp.int32),
            pltpu.VMEM((b_per_w, 2 * EMB), jnp.float32),
            pltpu.SemaphoreType.DMA,
        ],
    )
    def gather_k(table_hbm, idx_hbm, out_hbm, idx_v, rows_v, sem):
        wid = lax.axis_index("s") * nc + lax.axis_index("c")
        base = wid * b_per_w
        pltpu.sync_copy(idx_hbm.at[pl.ds(base, b_per_w)], idx_v)
        pltpu.async_copy(table_hbm.at[idx_v], rows_v, sem).wait()
        pltpu.sync_copy(rows_v, out_hbm.at[pl.ds(base, b_per_w)])

    return gather_k


_sc_gather = _make_sc_gather()


NV_BLK = 2048


def _proj_body(wv2_ref, half_ref, fcwt_ref, fcb_ref, out_ref, rhs_ref):
    @pl.when(pl.program_id(0) == 0)
    def _():
        wv2 = wv2_ref[...]
        emb = jnp.where(half_ref[...] > 0, wv2[:, EMB:], wv2[:, :EMB])
        s = jnp.sum(emb * emb, axis=1, keepdims=True)
        n = jnp.sqrt(s)
        scale = jnp.where(n > MAX_NORM, MAX_NORM / (n + 1e-7), 1.0)
        rhs_ref[:, :EMB] = emb * scale
        rhs_ref[:, EMB:] = jnp.ones((B, 1), jnp.float32)

    lhs = jnp.concatenate([fcwt_ref[...], fcb_ref[...]], axis=0)
    out_ref[...] = lax.dot_general(
        lhs, rhs_ref[...], (((0,), (1,)), ((), ())),
        preferred_element_type=jnp.float32,
    )


def _make_proj():
    grid = (pl.cdiv(VOCAB, NV_BLK),)
    return pl.pallas_call(
        _proj_body,
        grid=grid,
        in_specs=[
            pl.BlockSpec((B, 2 * EMB), lambda j: (0, 0)),
            pl.BlockSpec((B, 1), lambda j: (0, 0)),
            pl.BlockSpec((EMB, NV_BLK), lambda j: (0, j)),
            pl.BlockSpec((1, NV_BLK), lambda j: (0, j)),
        ],
        out_specs=pl.BlockSpec((NV_BLK, B), lambda j: (j, 0)),
        out_shape=jax.ShapeDtypeStruct((VOCAB, B), jnp.float32),
        scratch_shapes=[pltpu.VMEM((B, EMB + 1), jnp.float32)],
        compiler_params=pltpu.CompilerParams(
            dimension_semantics=("arbitrary",),
        ),
    )


_proj = _make_proj()


def kernel(word_vector, table, fc_w, fc_b):
    idx = word_vector[:, 0]
    table2 = _fmt(table)
    wv2 = _sc_gather(table2, idx >> 1)
    half = (idx & 1).reshape(B, 1)
    out_t = _proj(wv2, half, fc_w.T, fc_b.reshape(1, VOCAB))
    return out_t.T


# speedup vs baseline: 1.1913x; 1.1913x over previous
import functools

import jax
import jax.numpy as jnp
from jax import lax
from jax.experimental import pallas as pl
from jax.experimental.pallas import tpu as pltpu
from jax.experimental.pallas import tpu_sc as plsc

VOCAB = 100000
EMB = 64
B = 1024
MAX_NORM = 1.0


NVF = 2048


def _fmt_body(tt_ref, out_ref):
    t = jnp.transpose(tt_ref[...], (1, 0))
    t3 = t.reshape(NVF // 2, 2, EMB)
    out_ref[...] = jnp.concatenate([t3[:, 0, :], t3[:, 1, :]], axis=1)


def _make_fmt():
    grid = (pl.cdiv(VOCAB, NVF),)
    return pl.pallas_call(
        _fmt_body,
        grid=grid,
        in_specs=[pl.BlockSpec((EMB, NVF), lambda j: (0, j))],
        out_specs=pl.BlockSpec((NVF // 2, 2 * EMB), lambda j: (j, 0)),
        out_shape=jax.ShapeDtypeStruct((VOCAB // 2, 2 * EMB), jnp.float32),
        compiler_params=pltpu.CompilerParams(
            dimension_semantics=("arbitrary",),
        ),
    )


_fmt = _make_fmt()


def _make_sc_gather():
    info = plsc.get_sparse_core_info()
    nc, ns = info.num_cores, info.num_subcores
    nw = nc * ns
    b_per_w = B // nw
    mesh = plsc.VectorSubcoreMesh(core_axis_name="c", subcore_axis_name="s")

    @functools.partial(
        pl.kernel,
        mesh=mesh,
        out_type=jax.ShapeDtypeStruct((B, 2 * EMB), jnp.float32),
        scratch_types=[
            pltpu.VMEM((b_per_w,), jnp.int32),
            pltpu.VMEM((b_per_w, 2 * EMB), jnp.float32),
            pltpu.SemaphoreType.DMA,
        ],
    )
    def gather_k(table_hbm, idx_hbm, out_hbm, idx_v, rows_v, sem):
        wid = lax.axis_index("s") * nc + lax.axis_index("c")
        base = wid * b_per_w
        pltpu.sync_copy(idx_hbm.at[pl.ds(base, b_per_w)], idx_v)
        pltpu.async_copy(table_hbm.at[idx_v], rows_v, sem).wait()
        pltpu.sync_copy(rows_v, out_hbm.at[pl.ds(base, b_per_w)])

    return gather_k


_sc_gather = _make_sc_gather()


NV_BLK = 4096


def _proj_body(wv2_ref, half_ref, fcwt_ref, fcb_ref, out_ref, rhs_ref):
    @pl.when(pl.program_id(0) == 0)
    def _():
        wv2 = wv2_ref[...]
        emb = jnp.where(half_ref[...] > 0, wv2[:, EMB:], wv2[:, :EMB])
        s = jnp.sum(emb * emb, axis=1, keepdims=True)
        n = jnp.sqrt(s)
        scale = jnp.where(n > MAX_NORM, MAX_NORM / (n + 1e-7), 1.0)
        rhs_ref[:, :EMB] = emb * scale
        rhs_ref[:, EMB:] = jnp.ones((B, 1), jnp.float32)

    lhs = jnp.concatenate([fcwt_ref[...], fcb_ref[...]], axis=0)
    out_ref[...] = lax.dot_general(
        lhs, rhs_ref[...], (((0,), (1,)), ((), ())),
        preferred_element_type=jnp.float32,
    )


def _make_proj():
    grid = (pl.cdiv(VOCAB, NV_BLK),)
    return pl.pallas_call(
        _proj_body,
        grid=grid,
        in_specs=[
            pl.BlockSpec((B, 2 * EMB), lambda j: (0, 0)),
            pl.BlockSpec((B, 1), lambda j: (0, 0)),
            pl.BlockSpec((EMB, NV_BLK), lambda j: (0, j)),
            pl.BlockSpec((1, NV_BLK), lambda j: (0, j)),
        ],
        out_specs=pl.BlockSpec((NV_BLK, B), lambda j: (j, 0)),
        out_shape=jax.ShapeDtypeStruct((VOCAB, B), jnp.float32),
        scratch_shapes=[pltpu.VMEM((B, EMB + 1), jnp.float32)],
        compiler_params=pltpu.CompilerParams(
            dimension_semantics=("arbitrary",),
            vmem_limit_bytes=56 * 1024 * 1024,
        ),
    )


_proj = _make_proj()


def kernel(word_vector, table, fc_w, fc_b):
    idx = word_vector[:, 0]
    table2 = _fmt(table.T)
    wv2 = _sc_gather(table2, idx >> 1)
    half = (idx & 1).reshape(B, 1)
    out_t = _proj(wv2, half, fc_w.T, fc_b.reshape(1, VOCAB))
    return out_t.T
